# baseline (speedup 1.0000x reference)
import functools

import jax
import jax.numpy as jnp
from jax import lax
from jax.experimental import pallas as pl
from jax.experimental.pallas import tpu as pltpu
from jax.experimental.pallas import tpu_sc as plsc

_NUM_FEATURES = 26
_FEATURE_SIZE = 100000
_BATCH = 4096
_EMBED_DIM = 32
_NC = 2
_NS = 16
_NW = _NC * _NS
_BPW = _BATCH // _NW
_CB = 8
_HB = _CB // 2
_NCH = _BPW // _CB
_W16 = _NUM_FEATURES - 16


def _sc_body(x_hbm, table_hbm, out_hbm, x_v, rows_t, rows_s,
             gt_sem, gs_sem, st_sems, ss_sems):
    cid = lax.axis_index("c")
    sid = lax.axis_index("s")
    wid = sid * _NC + cid
    b0 = wid * _BPW

    pltpu.sync_copy(x_hbm.at[pl.ds(b0, _BPW)], x_v)

    def chunk(c, _):
        b = lax.rem(c, 2)

        @pl.when(c >= 2)
        def _wait_prev():
            pltpu.make_async_copy(
                rows_t.at[b], out_hbm.at[pl.ds(b0, _HB)],
                st_sems.at[b]).wait()
            pltpu.make_async_copy(
                rows_s.at[sid, b], out_hbm.at[pl.ds(b0, _HB)],
                ss_sems.at[b]).wait()

        def issue_t(jb, _):
            row = c * _CB + jb
            w0 = x_v[row, pl.ds(0, 16)]
            w1 = x_v[row, pl.ds(_W16, 16)]
            for jf in range(_NUM_FEATURES):
                xv = w0[jf] if jf < 16 else w1[jf - _W16]
                r = xv + jf * _FEATURE_SIZE
                pltpu.async_copy(
                    table_hbm.at[r], rows_t.at[b, jb, jf], gt_sem)
            return _

        def issue_s(jb, _):
            row = c * _CB + _HB + jb
            w0 = x_v[row, pl.ds(0, 16)]
            w1 = x_v[row, pl.ds(_W16, 16)]
            for jf in range(_NUM_FEATURES):
                xv = w0[jf] if jf < 16 else w1[jf - _W16]
                r = xv + jf * _FEATURE_SIZE
                pltpu.async_copy(
                    table_hbm.at[r], rows_s.at[sid, b, jb, jf], gs_sem)
            return _

        lax.fori_loop(0, _HB, issue_t, None)
        lax.fori_loop(0, _HB, issue_s, None)

        def drain_t(j, _):
            pltpu.make_async_copy(
                table_hbm.at[0], rows_t.at[b, 0, 0], gt_sem).wait()
            return _

        lax.fori_loop(0, _HB * _NUM_FEATURES, drain_t, None, unroll=8)

        def drain_s(j, _):
            pltpu.make_async_copy(
                table_hbm.at[0], rows_s.at[sid, b, 0, 0], gs_sem).wait()
            return _

        lax.fori_loop(0, _HB * _NUM_FEATURES, drain_s, None, unroll=8)

        pltpu.async_copy(
            rows_t.at[b], out_hbm.at[pl.ds(b0 + c * _CB, _HB)],
            st_sems.at[b])
        pltpu.async_copy(
            rows_s.at[sid, b], out_hbm.at[pl.ds(b0 + c * _CB + _HB, _HB)],
            ss_sems.at[b])
        return _

    lax.fori_loop(0, _NCH, chunk, None)

    for b in range(2):
        pltpu.make_async_copy(
            rows_t.at[b], out_hbm.at[pl.ds(b0, _HB)], st_sems.at[b]).wait()
        pltpu.make_async_copy(
            rows_s.at[sid, b], out_hbm.at[pl.ds(b0, _HB)],
            ss_sems.at[b]).wait()


@jax.jit
def kernel(x, table):
    x = x.astype(jnp.int32)
    mesh = plsc.VectorSubcoreMesh(
        core_axis_name="c", subcore_axis_name="s",
        num_cores=_NC, num_subcores=_NS,
    )
    return pl.kernel(
        _sc_body,
        out_type=jax.ShapeDtypeStruct(
            (_BATCH, _NUM_FEATURES, _EMBED_DIM), jnp.float32),
        mesh=mesh,
        scratch_types=[
            pltpu.VMEM((_BPW, _NUM_FEATURES), jnp.int32),
            pltpu.VMEM((2, _HB, _NUM_FEATURES, _EMBED_DIM), jnp.float32),
            pltpu.VMEM_SHARED(
                (_NS, 2, _HB, _NUM_FEATURES, _EMBED_DIM), jnp.float32),
            pltpu.SemaphoreType.DMA,
            pltpu.SemaphoreType.DMA,
            pltpu.SemaphoreType.DMA((2,)),
            pltpu.SemaphoreType.DMA((2,)),
        ],
    )(x, table)


# speedup vs baseline: 1.0815x; 1.0815x over previous
import functools

import jax
import jax.numpy as jnp
from jax import lax
from jax.experimental import pallas as pl
from jax.experimental.pallas import tpu as pltpu
from jax.experimental.pallas import tpu_sc as plsc

_NUM_FEATURES = 26
_FEATURE_SIZE = 100000
_BATCH = 4096
_EMBED_DIM = 32
_NC = 2
_NS = 16
_NW = _NC * _NS
_BPW = _BATCH // _NW
_CB = 8
_NCH = _BPW // _CB
_NSEM = 4
_W16 = _NUM_FEATURES - 16


def _sc_body(x_hbm, table_hbm, out_hbm, x_v, rows_v, g_sems, s_sems):
    wid = lax.axis_index("s") * _NC + lax.axis_index("c")
    b0 = wid * _BPW

    pltpu.sync_copy(x_hbm.at[pl.ds(b0, _BPW)], x_v)

    def issue(c, b):
        def body(jb, _):
            row = c * _CB + jb
            w0 = x_v[row, pl.ds(0, 16)]
            w1 = x_v[row, pl.ds(_W16, 16)]
            for jf in range(_NUM_FEATURES):
                xv = w0[jf] if jf < 16 else w1[jf - _W16]
                r = xv + jf * _FEATURE_SIZE
                pltpu.async_copy(
                    table_hbm.at[r], rows_v.at[b, jb, jf],
                    g_sems.at[b, jf % _NSEM])
            return _

        lax.fori_loop(0, _CB, body, None)

    def drain_and_scatter(c, b):
        for s in range(_NSEM):
            n_waits = _CB * len(
                [f for f in range(_NUM_FEATURES) if f % _NSEM == s])

            def drain(j, _, s=s):
                pltpu.make_async_copy(
                    table_hbm.at[0], rows_v.at[b, 0, 0],
                    g_sems.at[b, s]).wait()
                return _

            lax.fori_loop(0, n_waits, drain, None, unroll=8)

        pltpu.async_copy(
            rows_v.at[b], out_hbm.at[pl.ds(b0 + c * _CB, _CB)], s_sems.at[b])

    def chunk(c, _):
        b = lax.rem(c, 2)

        @pl.when(c >= 2)
        def _wait_prev():
            pltpu.make_async_copy(
                rows_v.at[b], out_hbm.at[pl.ds(b0, _CB)], s_sems.at[b]
            ).wait()

        issue(c, b)

        @pl.when(c >= 1)
        def _finish_prev():
            drain_and_scatter(c - 1, 1 - b)

        return _

    lax.fori_loop(0, _NCH, chunk, None)
    drain_and_scatter(_NCH - 1, lax.rem(_NCH - 1, 2))

    for b in range(2):
        pltpu.make_async_copy(
            rows_v.at[b], out_hbm.at[pl.ds(b0, _CB)], s_sems.at[b]).wait()


@jax.jit
def kernel(x, table):
    x = x.astype(jnp.int32)
    mesh = plsc.VectorSubcoreMesh(
        core_axis_name="c", subcore_axis_name="s",
        num_cores=_NC, num_subcores=_NS,
    )
    return pl.kernel(
        _sc_body,
        out_type=jax.ShapeDtypeStruct(
            (_BATCH, _NUM_FEATURES, _EMBED_DIM), jnp.float32),
        mesh=mesh,
        scratch_types=[
            pltpu.VMEM((_BPW, _NUM_FEATURES), jnp.int32),
            pltpu.VMEM((2, _CB, _NUM_FEATURES, _EMBED_DIM), jnp.float32),
            pltpu.SemaphoreType.DMA((2, _NSEM)),
            pltpu.SemaphoreType.DMA((2,)),
        ],
    )(x, table)
